# baseline (speedup 1.0000x reference)
import functools

import jax
import jax.numpy as jnp
from jax import lax
from jax.experimental import pallas as pl
from jax.experimental.pallas import tpu as pltpu
from jax.experimental.pallas import tpu_sc as plsc

_N, _C, _J = 65536, 80, 600
_NW = 32
_RPW = _N // _NW
_RB = 64
_NCH = _RPW // _RB
_NG = _J // 16
_JP = 608

_mesh = plsc.VectorSubcoreMesh(core_axis_name="c", subcore_axis_name="s")


def _sc_body(x_hbm, idx_hbm, out_hbm, idx_v, in_v, out_v):
    cid = lax.axis_index("c")
    sid = lax.axis_index("s")
    wid = sid * 2 + cid
    row0 = wid * _RPW

    idx_v[pl.ds(592, 16)] = jnp.zeros((16,), jnp.int32)
    pltpu.sync_copy(idx_hbm, idx_v.at[pl.ds(0, _J)])

    iota = lax.iota(jnp.int32, 16)
    tail_mask = iota < (_J - _NG * 16)

    def column_pass(g, mask):
        src0 = idx_v[pl.ds(g * 16, 16)]
        dst0 = iota + (g * 16)

        @plsc.parallel_loop(0, _RB, 1, unroll=4, carry=(src0, dst0))
        def _(r, c):
            src, dst = c
            v = plsc.load_gather(in_v, [src], mask=mask)
            plsc.store_scatter(out_v, [dst], v, mask=mask)
            return (src + _C, dst + _J)

    def chunk_body(k, _):
        r0 = row0 + k * _RB
        pltpu.sync_copy(x_hbm.at[pl.ds(r0 * _C, _RB * _C)], in_v)
        for g in range(_NG):
            column_pass(g, None)
        column_pass(_NG, tail_mask)
        pltpu.sync_copy(out_v, out_hbm.at[pl.ds(r0 * _J, _RB * _J)])
        return 0

    lax.fori_loop(0, _NCH, chunk_body, 0)


_sc_call = functools.partial(
    pl.kernel,
    out_type=jax.ShapeDtypeStruct((_N * _J,), jnp.float32),
    mesh=_mesh,
    compiler_params=pltpu.CompilerParams(
        needs_layout_passes=False, disable_bounds_checks=True),
    scratch_types=[
        pltpu.VMEM((_JP,), jnp.int32),
        pltpu.VMEM((_RB * _C,), jnp.float32),
        pltpu.VMEM((_RB * _J,), jnp.float32),
    ],
)(_sc_body)


def kernel(o_conf, obj2hoi):
    out = _sc_call(o_conf.reshape(_N * _C), obj2hoi.astype(jnp.int32))
    return out.reshape(_N, _J)


# speedup vs baseline: 1.5560x; 1.5560x over previous
import functools

import jax
import jax.numpy as jnp
from jax import lax
from jax.experimental import pallas as pl
from jax.experimental.pallas import tpu as pltpu
from jax.experimental.pallas import tpu_sc as plsc

_N, _C, _J = 65536, 80, 600
_NW = 32
_RPW = _N // _NW
_RB = 64
_NCH = _RPW // _RB
_NG = _J // 16
_JP = 608

_mesh = plsc.VectorSubcoreMesh(core_axis_name="c", subcore_axis_name="s")


def _sc_body(x_hbm, idx_hbm, out_hbm, idx_v, in2_v, in_v, out_v):
    cid = lax.axis_index("c")
    sid = lax.axis_index("s")
    wid = sid * 2 + cid
    row0 = wid * _RPW

    idx_v[pl.ds(592, 16)] = jnp.zeros((16,), jnp.int32)
    pltpu.sync_copy(idx_hbm, idx_v.at[pl.ds(0, _J)])

    iota = lax.iota(jnp.int32, 16)
    ones = jnp.ones((16,), jnp.int32)
    tail_mask = iota < (_J - _NG * 16)
    tail_j = iota + (_NG * 16)

    def column_pass(g):
        src0 = idx_v[pl.ds(g * 16, 16)]

        @plsc.parallel_loop(0, _RB, 1, unroll=8, carry=src0)
        def _(r, src):
            v = plsc.load_gather(in_v, [src])
            out_v[r, pl.ds(g * 16, 16)] = v
            return src + _C

    def tail_pass():
        src0 = idx_v[pl.ds(_NG * 16, 16)]

        @plsc.parallel_loop(0, _RB, 1, unroll=4, carry=(src0, jnp.zeros((16,), jnp.int32)))
        def _(r, c):
            src, r_vec = c
            v = plsc.load_gather(in_v, [src], mask=tail_mask)
            plsc.store_scatter(out_v, [r_vec, tail_j], v, mask=tail_mask)
            return (src + _C, r_vec + ones)

    def chunk_body(k, _):
        r0 = row0 + k * _RB
        pltpu.sync_copy(x_hbm.at[pl.ds(r0, _RB)], in2_v)

        @plsc.parallel_loop(0, _RB, 1, unroll=2)
        def _(r):
            for t in range(_C // 16):
                in_v[pl.ds(r * _C + t * 16, 16)] = in2_v[r, pl.ds(t * 16, 16)]

        for g in range(_NG):
            column_pass(g)
        tail_pass()
        pltpu.sync_copy(out_v, out_hbm.at[pl.ds(r0, _RB)])
        return 0

    lax.fori_loop(0, _NCH, chunk_body, 0)


_sc_call = functools.partial(
    pl.kernel,
    out_type=jax.ShapeDtypeStruct((_N, _J), jnp.float32),
    mesh=_mesh,
    compiler_params=pltpu.CompilerParams(
        needs_layout_passes=False, disable_bounds_checks=True),
    scratch_types=[
        pltpu.VMEM((_JP,), jnp.int32),
        pltpu.VMEM((_RB, _C), jnp.float32),
        pltpu.VMEM((_RB * _C,), jnp.float32),
        pltpu.VMEM((_RB, _J), jnp.float32),
    ],
)(_sc_body)


def kernel(o_conf, obj2hoi):
    return _sc_call(o_conf, obj2hoi.astype(jnp.int32))
